# baseline (speedup 1.0000x reference)
import jax
import jax.numpy as jnp
from jax.experimental import pallas as pl

B = 8
N = 2048
K = 40
EMB = 1024
SEG = 50


def _bn(y, g, b):
    axes = (0,) + tuple(range(2, y.ndim))
    mean = jnp.mean(y, axis=axes, keepdims=True)
    var = jnp.var(y, axis=axes, keepdims=True)
    yn = (y - mean) / jnp.sqrt(var + 1e-5)
    shape = [1, -1] + [1] * (y.ndim - 2)
    return yn * g.reshape(shape) + b.reshape(shape)


def _lrelu(y):
    return jnp.where(y >= 0, y, 0.2 * y)


def _conv2(y, W, g, b):
    return _lrelu(_bn(jnp.einsum('oc,bcnk->bonk', W, y), g, b))


def _conv1(y, W, g, b):
    return _lrelu(_bn(jnp.einsum('oc,bcn->bon', W, y), g, b))


def _knn(x, k):
    inner = -2.0 * jnp.einsum('bcn,bcm->bnm', x, x)
    xx = jnp.sum(x * x, axis=1, keepdims=True)
    pd = -xx - inner - jnp.transpose(xx, (0, 2, 1))
    return jax.lax.top_k(pd, k)[1]


def _graph_feature(x, k):
    b, c, n = x.shape
    idx = _knn(x, k)
    xt = jnp.transpose(x, (0, 2, 1))
    bidx = jnp.arange(b)[:, None, None]
    feature = xt[bidx, idx]
    xc = jnp.broadcast_to(xt[:, :, None, :], (b, n, k, c))
    out = jnp.concatenate([feature - xc, xc], axis=3)
    return jnp.transpose(out, (0, 3, 1, 2))


def _pool(xyz, feature, Wp, npts):
    b = feature.shape[0]
    scores = jax.nn.sigmoid(jnp.einsum('oc,bcn->bon', Wp, feature))[:, 0, :]
    vals, idx = jax.lax.top_k(scores, npts)
    bidx = jnp.arange(b)[:, None]
    xyz_new = jnp.transpose(jnp.transpose(xyz, (0, 2, 1))[bidx, idx], (0, 2, 1))
    feat_new = jnp.transpose(jnp.transpose(feature, (0, 2, 1))[bidx, idx], (0, 2, 1))
    feat_new = feat_new * vals[:, None, :]
    return xyz_new, feat_new, xyz_new


def _unpool(xyz, unknown_xyz, features):
    b = xyz.shape[0]
    diff = unknown_xyz[:, :, :, None] - xyz[:, :, None, :]
    diff_norm = jnp.sum(diff * diff, axis=1)
    neg_dist, nn_idx = jax.lax.top_k(-diff_norm, 3)
    weight = jax.nn.softmax(-neg_dist, axis=-1)
    featT = jnp.transpose(features, (0, 2, 1))
    bidx = jnp.arange(b)[:, None, None]
    grouped = featT[bidx, nn_idx]
    out = jnp.sum(weight[..., None] * grouped, axis=2)
    return jnp.transpose(out, (0, 2, 1))


def _identity_kernel(x_ref, o_ref):
    o_ref[...] = x_ref[...]


def _pallas_identity(x):
    return pl.pallas_call(
        _identity_kernel,
        out_shape=jax.ShapeDtypeStruct(x.shape, x.dtype),
    )(x)


def kernel(x, l, params):
    p = params
    xyz = x
    y = _graph_feature(x, K)
    y = _conv2(y, p['conv1'], p['bn1_g'], p['bn1_b'])
    y = _conv2(y, p['conv2'], p['bn2_g'], p['bn2_b'])
    x1 = jnp.max(y, axis=-1)
    x1 = _pallas_identity(x1)
    node1, nf1, node1_s = _pool(xyz, x1, p['pool1'], N // 4)
    y = _graph_feature(nf1, K // 2)
    y = _conv2(y, p['conv3'], p['bn3_g'], p['bn3_b'])
    y = _conv2(y, p['conv4'], p['bn4_g'], p['bn4_b'])
    x2 = jnp.max(y, axis=-1)
    node2, nf2, node2_s = _pool(node1_s, x2, p['pool2'], N // 16)
    y = _graph_feature(nf2, K // 4)
    y = _conv2(y, p['conv5'], p['bn5_g'], p['bn5_b'])
    x3 = jnp.max(y, axis=-1)
    node3, nf3, node3_s = _pool(node2_s, x3, p['pool3'], N // 64)
    y = _graph_feature(nf3, K // 8)
    y = _conv2(y, p['conv6'], p['bn6_g'], p['bn6_b'])
    x4 = jnp.max(y, axis=-1)
    x1_t = jnp.max(x1, axis=-1, keepdims=True)
    x2_t = jnp.max(x2, axis=-1, keepdims=True)
    x3_t = jnp.max(x3, axis=-1, keepdims=True)
    x4_t = jnp.max(x4, axis=-1, keepdims=True)
    y = jnp.concatenate([x1_t, x2_t, x3_t, x4_t], axis=1)
    y = _conv1(y, p['conv6_m'], p['bn6_m_g'], p['bn6_m_b'])
    lv = l.reshape(l.shape[0], -1, 1)
    lv = _conv1(lv, p['conv7'], p['bn7_g'], p['bn7_b'])
    y = jnp.concatenate([y, lv], axis=1)
    y = _conv1(y, p['conv8'], p['bn8_g'], p['bn8_b'])
    y = jnp.tile(y, (1, 1, x4.shape[-1]))
    y = jnp.concatenate([y, x4], axis=1)
    y = _conv1(y, p['conv9'], p['bn9_g'], p['bn9_b'])
    y = _unpool(node3_s, node2_s, y)
    y = jnp.concatenate([y, x3], axis=1)
    y = _conv1(y, p['conv10'], p['bn10_g'], p['bn10_b'])
    y = _unpool(node2_s, node1_s, y)
    y = jnp.concatenate([y, x2], axis=1)
    y = _conv1(y, p['conv11'], p['bn11_g'], p['bn11_b'])
    y = _unpool(node1_s, xyz, y)
    y = jnp.concatenate([y, x1], axis=1)
    y = _conv1(y, p['conv12'], p['bn12_g'], p['bn12_b'])
    y = jnp.einsum('oc,bcn->bon', p['conv13'], y)
    return (y, node1, node2, node3, node1_s, node2_s)


# speedup vs baseline: 1.4030x; 1.4030x over previous
import functools

import jax
import jax.numpy as jnp
from jax.experimental import pallas as pl
from jax.experimental.pallas import tpu as pltpu

B = 8
N = 2048
K = 40
EMB = 1024
SEG = 50

_NEG = -3e38
_BIGI = 2 ** 30


def _topk_kernel(pd_in_ref, idx_ref, pd_ref, *, k, n, tile):
    pd_ref[...] = pd_in_ref[0]
    lane = jax.lax.broadcasted_iota(jnp.int32, (tile, n), 1)
    klane = jax.lax.broadcasted_iota(jnp.int32, (tile, k), 1)

    def body(kk, idxbuf):
        pd = pd_ref[...]
        m = jnp.max(pd, axis=1, keepdims=True)
        j = jnp.min(jnp.where(pd == m, lane, _BIGI), axis=1, keepdims=True)
        idxbuf = jnp.where(klane == kk, j, idxbuf)
        pd_ref[...] = jnp.where(lane == j, _NEG, pd)
        return idxbuf

    idxbuf = jnp.zeros((tile, k), jnp.int32)
    idx_ref[0] = jax.lax.fori_loop(0, k, body, idxbuf)


def _topk_rows(pd, k):
    b, n, m = pd.shape
    tile = min(n, 256)
    grid = (b, n // tile)
    return pl.pallas_call(
        functools.partial(_topk_kernel, k=k, n=m, tile=tile),
        grid=grid,
        in_specs=[pl.BlockSpec((1, tile, m), lambda bi, ti: (bi, ti, 0))],
        out_specs=pl.BlockSpec((1, tile, k), lambda bi, ti: (bi, ti, 0)),
        out_shape=jax.ShapeDtypeStruct((b, n, k), jnp.int32),
        scratch_shapes=[pltpu.VMEM((tile, m), jnp.float32)],
    )(pd)


def _knn(x, k):
    inner = -2.0 * jnp.einsum('bcn,bcm->bnm', x, x)
    xx = jnp.sum(x * x, axis=1, keepdims=True)
    pd = -xx - inner - jnp.transpose(xx, (0, 2, 1))
    return _topk_rows(pd, k)


def _graph_feature(x, k):
    b, c, n = x.shape
    idx = _knn(x, k)
    xt = jnp.transpose(x, (0, 2, 1))
    bidx = jnp.arange(b)[:, None, None]
    feature = xt[bidx, idx]
    xc = jnp.broadcast_to(xt[:, :, None, :], (b, n, k, c))
    out = jnp.concatenate([feature - xc, xc], axis=3)
    return jnp.transpose(out, (0, 3, 1, 2))


def _pool_kernel(sc_ref, feat_ref, xyz_ref, node_ref, fnew_ref, *, n, p, nb):
    scores = sc_ref[...]
    lane = jax.lax.broadcasted_iota(jnp.int32, (nb, n), 1)
    plane = jax.lax.broadcasted_iota(jnp.int32, (nb, p), 1)

    def body(kk, carry):
        s, idxb, valb = carry
        m = jnp.max(s, axis=1, keepdims=True)
        j = jnp.min(jnp.where(s == m, lane, _BIGI), axis=1, keepdims=True)
        idxb = jnp.where(plane == kk, j, idxb)
        valb = jnp.where(plane == kk, m, valb)
        s = jnp.where(lane == j, _NEG, s)
        return s, idxb, valb

    _, idxb, valb = jax.lax.fori_loop(
        0, p, body,
        (scores, jnp.zeros((nb, p), jnp.int32), jnp.zeros((nb, p), jnp.float32)),
    )

    riota = jax.lax.broadcasted_iota(jnp.int32, (n, p), 0)
    for bi in range(nb):
        sel = jnp.where(riota == idxb[bi][None, :], 1.0, 0.0)
        fg = jnp.dot(feat_ref[bi], sel, preferred_element_type=jnp.float32,
                     precision=jax.lax.Precision.HIGHEST)
        fnew_ref[bi] = fg * valb[bi][None, :]
        node_ref[bi] = jnp.dot(xyz_ref[bi], sel,
                               preferred_element_type=jnp.float32,
                     precision=jax.lax.Precision.HIGHEST)


def _pool_pallas(xyz8, feature, wp, p):
    b, c, n = feature.shape
    scores = jax.nn.sigmoid(jnp.einsum('oc,bcn->bon', wp, feature))[:, 0, :]
    node8, fnew = pl.pallas_call(
        functools.partial(_pool_kernel, n=n, p=p, nb=b),
        in_specs=[
            pl.BlockSpec((b, n), lambda: (0, 0)),
            pl.BlockSpec((b, c, n), lambda: (0, 0, 0)),
            pl.BlockSpec((b, 8, n), lambda: (0, 0, 0)),
        ],
        out_specs=[
            pl.BlockSpec((b, 8, p), lambda: (0, 0, 0)),
            pl.BlockSpec((b, c, p), lambda: (0, 0, 0)),
        ],
        out_shape=[
            jax.ShapeDtypeStruct((b, 8, p), jnp.float32),
            jax.ShapeDtypeStruct((b, c, p), jnp.float32),
        ],
    )(scores, feature, xyz8)
    return node8, fnew


def _unpool_kernel(neg_ref, featT_ref, out_ref, *, m, tile):
    neg = neg_ref[0]
    lane = jax.lax.broadcasted_iota(jnp.int32, (tile, m), 1)

    ms, js = [], []
    for _ in range(3):
        mv = jnp.max(neg, axis=1, keepdims=True)
        jv = jnp.min(jnp.where(neg == mv, lane, _BIGI), axis=1, keepdims=True)
        ms.append(mv)
        js.append(jv)
        neg = jnp.where(lane == jv, _NEG, neg)

    mx = -ms[2]
    es = [jnp.exp(-mv - mx) for mv in ms]
    tot = (es[0] + es[1]) + es[2]
    sel = jnp.zeros((tile, m), jnp.float32)
    for t in range(3):
        sel = sel + jnp.where(lane == js[t], es[t] / tot, 0.0)
    out_ref[0] = jnp.dot(sel, featT_ref[0], preferred_element_type=jnp.float32,
                     precision=jax.lax.Precision.HIGHEST)


def _unpool(xyz, unknown_xyz, features):
    b, _, m = xyz.shape
    n = unknown_xyz.shape[2]
    c = features.shape[1]
    diff = unknown_xyz[:, :, :, None] - xyz[:, :, None, :]
    neg = -jnp.sum(diff * diff, axis=1)
    featT = jnp.transpose(features, (0, 2, 1))
    tile = min(n, 256)
    grid = (b, n // tile)
    outT = pl.pallas_call(
        functools.partial(_unpool_kernel, m=m, tile=tile),
        grid=grid,
        in_specs=[
            pl.BlockSpec((1, tile, m), lambda bi, ti: (bi, ti, 0)),
            pl.BlockSpec((1, m, c), lambda bi, ti: (bi, 0, 0)),
        ],
        out_specs=pl.BlockSpec((1, tile, c), lambda bi, ti: (bi, ti, 0)),
        out_shape=jax.ShapeDtypeStruct((b, n, c), jnp.float32),
    )(neg, featT)
    return jnp.transpose(outT, (0, 2, 1))


def _bn(y, g, b):
    axes = (0,) + tuple(range(2, y.ndim))
    mean = jnp.mean(y, axis=axes, keepdims=True)
    var = jnp.var(y, axis=axes, keepdims=True)
    yn = (y - mean) / jnp.sqrt(var + 1e-5)
    shape = [1, -1] + [1] * (y.ndim - 2)
    return yn * g.reshape(shape) + b.reshape(shape)


def _lrelu(y):
    return jnp.where(y >= 0, y, 0.2 * y)


def _conv2(y, W, g, b):
    return _lrelu(_bn(jnp.einsum('oc,bcnk->bonk', W, y), g, b))


def _conv1(y, W, g, b):
    return _lrelu(_bn(jnp.einsum('oc,bcn->bon', W, y), g, b))


def kernel(x, l, params):
    p = params
    xyz = x
    xyz8 = jnp.pad(xyz, ((0, 0), (0, 5), (0, 0)))

    y = _graph_feature(x, K)
    y = _conv2(y, p['conv1'], p['bn1_g'], p['bn1_b'])
    y = _conv2(y, p['conv2'], p['bn2_g'], p['bn2_b'])
    x1 = jnp.max(y, axis=-1)

    node1_8, nf1 = _pool_pallas(xyz8, x1, p['pool1'], N // 4)
    node1 = node1_8[:, :3, :]

    y = _graph_feature(nf1, K // 2)
    y = _conv2(y, p['conv3'], p['bn3_g'], p['bn3_b'])
    y = _conv2(y, p['conv4'], p['bn4_g'], p['bn4_b'])
    x2 = jnp.max(y, axis=-1)

    node2_8, nf2 = _pool_pallas(node1_8, x2, p['pool2'], N // 16)
    node2 = node2_8[:, :3, :]

    y = _graph_feature(nf2, K // 4)
    y = _conv2(y, p['conv5'], p['bn5_g'], p['bn5_b'])
    x3 = jnp.max(y, axis=-1)

    node3_8, nf3 = _pool_pallas(node2_8, x3, p['pool3'], N // 64)
    node3 = node3_8[:, :3, :]

    y = _graph_feature(nf3, K // 8)
    y = _conv2(y, p['conv6'], p['bn6_g'], p['bn6_b'])
    x4 = jnp.max(y, axis=-1)

    x1_t = jnp.max(x1, axis=-1, keepdims=True)
    x2_t = jnp.max(x2, axis=-1, keepdims=True)
    x3_t = jnp.max(x3, axis=-1, keepdims=True)
    x4_t = jnp.max(x4, axis=-1, keepdims=True)
    y = jnp.concatenate([x1_t, x2_t, x3_t, x4_t], axis=1)
    y = _conv1(y, p['conv6_m'], p['bn6_m_g'], p['bn6_m_b'])
    lv = l.reshape(l.shape[0], -1, 1)
    lv = _conv1(lv, p['conv7'], p['bn7_g'], p['bn7_b'])
    y = jnp.concatenate([y, lv], axis=1)
    y = _conv1(y, p['conv8'], p['bn8_g'], p['bn8_b'])
    y = jnp.tile(y, (1, 1, x4.shape[-1]))
    y = jnp.concatenate([y, x4], axis=1)
    y = _conv1(y, p['conv9'], p['bn9_g'], p['bn9_b'])

    node1s = node1_8[:, :3, :]
    node2s = node2_8[:, :3, :]
    node3s = node3_8[:, :3, :]
    y = _unpool(node3s, node2s, y)
    y = jnp.concatenate([y, x3], axis=1)
    y = _conv1(y, p['conv10'], p['bn10_g'], p['bn10_b'])

    y = _unpool(node2s, node1s, y)
    y = jnp.concatenate([y, x2], axis=1)
    y = _conv1(y, p['conv11'], p['bn11_g'], p['bn11_b'])

    y = _unpool(node1s, xyz, y)
    y = jnp.concatenate([y, x1], axis=1)
    y = _conv1(y, p['conv12'], p['bn12_g'], p['bn12_b'])
    y = jnp.einsum('oc,bcn->bon', p['conv13'], y)
    return (y, node1, node2, node3, node1, node2)


# speedup vs baseline: 3.5421x; 2.5247x over previous
import functools

import jax
import jax.numpy as jnp
from jax.experimental import pallas as pl
from jax.experimental.pallas import tpu as pltpu

B = 8
N = 2048
K = 40
EMB = 1024
SEG = 50

_NEG = -3e38
_BIGI = 2 ** 30


def _topk_kernel(pd_in_ref, idx_ref, pd_ref, *, k, n, tile):
    pd_ref[...] = pd_in_ref[0]
    lane = jax.lax.broadcasted_iota(jnp.int32, (tile, n), 1)
    klane = jax.lax.broadcasted_iota(jnp.int32, (tile, k), 1)

    def body(kk, idxbuf):
        pd = pd_ref[...]
        m = jnp.max(pd, axis=1, keepdims=True)
        j = jnp.min(jnp.where(pd == m, lane, _BIGI), axis=1, keepdims=True)
        idxbuf = jnp.where(klane == kk, j, idxbuf)
        pd_ref[...] = jnp.where(lane == j, _NEG, pd)
        return idxbuf

    idxbuf = jnp.zeros((tile, k), jnp.int32)
    idx_ref[0] = jax.lax.fori_loop(0, k, body, idxbuf)


def _topk_rows(pd, k):
    b, n, m = pd.shape
    tile = min(n, 256)
    grid = (b, n // tile)
    return pl.pallas_call(
        functools.partial(_topk_kernel, k=k, n=m, tile=tile),
        grid=grid,
        in_specs=[pl.BlockSpec((1, tile, m), lambda bi, ti: (bi, ti, 0))],
        out_specs=pl.BlockSpec((1, tile, k), lambda bi, ti: (bi, ti, 0)),
        out_shape=jax.ShapeDtypeStruct((b, n, k), jnp.int32),
        scratch_shapes=[pltpu.VMEM((tile, m), jnp.float32)],
    )(pd)


def _knn(x, k):
    inner = -2.0 * jnp.einsum('bcn,bcm->bnm', x, x)
    xx = jnp.sum(x * x, axis=1, keepdims=True)
    pd = -xx - inner - jnp.transpose(xx, (0, 2, 1))
    return _topk_rows(pd, k)


def _gf_kernel(x_ref, idxT_ref, out_ref, *, n, tile, c8):
    ti = pl.program_id(2)
    x_all = x_ref[0]
    idxrow = idxT_ref[0, 0]
    siota = jax.lax.broadcasted_iota(jnp.int32, (n, tile), 0)
    ohT = jnp.where(siota == idxrow, 1.0, 0.0)
    g = jnp.dot(x_all, ohT, preferred_element_type=jnp.float32,
                precision=jax.lax.Precision.HIGHEST)
    if n == tile:
        ctr = x_all
    else:
        ctr = x_ref[0, :, pl.ds(pl.multiple_of(ti * tile, tile), tile)]
    out_ref[0, 0] = jnp.concatenate([g - ctr, ctr], axis=0)


def _graph_feature(x, k):
    b, c, n = x.shape
    idx = _knn(x, k)
    c8 = max(8, ((c + 7) // 8) * 8)
    xp = jnp.pad(x, ((0, 0), (0, c8 - c), (0, 0))) if c8 != c else x
    idxT = jnp.transpose(idx, (0, 2, 1))[:, :, None, :]
    tile = min(n, 256)
    grid = (b, k, n // tile)
    out = pl.pallas_call(
        functools.partial(_gf_kernel, n=n, tile=tile, c8=c8),
        grid=grid,
        in_specs=[
            pl.BlockSpec((1, c8, n), lambda bi, ki, ti: (bi, 0, 0)),
            pl.BlockSpec((1, 1, 1, tile), lambda bi, ki, ti: (bi, ki, 0, ti)),
        ],
        out_specs=pl.BlockSpec((1, 1, 2 * c8, tile),
                               lambda bi, ki, ti: (bi, ki, 0, ti)),
        out_shape=jax.ShapeDtypeStruct((b, k, 2 * c8, n), jnp.float32),
    )(xp, idxT)
    outT = jnp.transpose(out, (0, 2, 3, 1))
    if c8 != c:
        outT = jnp.concatenate([outT[:, :c], outT[:, c8:c8 + c]], axis=1)
    return outT


def _pool_kernel(sc_ref, feat_ref, xyz_ref, node_ref, fnew_ref, *, n, p, nb):
    scores = sc_ref[...]
    lane = jax.lax.broadcasted_iota(jnp.int32, (nb, n), 1)
    plane = jax.lax.broadcasted_iota(jnp.int32, (nb, p), 1)

    def body(kk, carry):
        s, idxb, valb = carry
        m = jnp.max(s, axis=1, keepdims=True)
        j = jnp.min(jnp.where(s == m, lane, _BIGI), axis=1, keepdims=True)
        idxb = jnp.where(plane == kk, j, idxb)
        valb = jnp.where(plane == kk, m, valb)
        s = jnp.where(lane == j, _NEG, s)
        return s, idxb, valb

    _, idxb, valb = jax.lax.fori_loop(
        0, p, body,
        (scores, jnp.zeros((nb, p), jnp.int32), jnp.zeros((nb, p), jnp.float32)),
    )

    riota = jax.lax.broadcasted_iota(jnp.int32, (n, p), 0)
    for bi in range(nb):
        sel = jnp.where(riota == idxb[bi][None, :], 1.0, 0.0)
        fg = jnp.dot(feat_ref[bi], sel, preferred_element_type=jnp.float32,
                     precision=jax.lax.Precision.HIGHEST)
        fnew_ref[bi] = fg * valb[bi][None, :]
        node_ref[bi] = jnp.dot(xyz_ref[bi], sel,
                               preferred_element_type=jnp.float32,
                     precision=jax.lax.Precision.HIGHEST)


def _pool_pallas(xyz8, feature, wp, p):
    b, c, n = feature.shape
    scores = jax.nn.sigmoid(jnp.einsum('oc,bcn->bon', wp, feature))[:, 0, :]
    node8, fnew = pl.pallas_call(
        functools.partial(_pool_kernel, n=n, p=p, nb=b),
        in_specs=[
            pl.BlockSpec((b, n), lambda: (0, 0)),
            pl.BlockSpec((b, c, n), lambda: (0, 0, 0)),
            pl.BlockSpec((b, 8, n), lambda: (0, 0, 0)),
        ],
        out_specs=[
            pl.BlockSpec((b, 8, p), lambda: (0, 0, 0)),
            pl.BlockSpec((b, c, p), lambda: (0, 0, 0)),
        ],
        out_shape=[
            jax.ShapeDtypeStruct((b, 8, p), jnp.float32),
            jax.ShapeDtypeStruct((b, c, p), jnp.float32),
        ],
    )(scores, feature, xyz8)
    return node8, fnew


def _unpool_kernel(neg_ref, featT_ref, out_ref, *, m, tile):
    neg = neg_ref[0]
    lane = jax.lax.broadcasted_iota(jnp.int32, (tile, m), 1)

    ms, js = [], []
    for _ in range(3):
        mv = jnp.max(neg, axis=1, keepdims=True)
        jv = jnp.min(jnp.where(neg == mv, lane, _BIGI), axis=1, keepdims=True)
        ms.append(mv)
        js.append(jv)
        neg = jnp.where(lane == jv, _NEG, neg)

    mx = -ms[2]
    es = [jnp.exp(-mv - mx) for mv in ms]
    tot = (es[0] + es[1]) + es[2]
    sel = jnp.zeros((tile, m), jnp.float32)
    for t in range(3):
        sel = sel + jnp.where(lane == js[t], es[t] / tot, 0.0)
    out_ref[0] = jnp.dot(sel, featT_ref[0], preferred_element_type=jnp.float32,
                     precision=jax.lax.Precision.HIGHEST)


def _unpool(xyz, unknown_xyz, features):
    b, _, m = xyz.shape
    n = unknown_xyz.shape[2]
    c = features.shape[1]
    diff = unknown_xyz[:, :, :, None] - xyz[:, :, None, :]
    neg = -jnp.sum(diff * diff, axis=1)
    featT = jnp.transpose(features, (0, 2, 1))
    tile = min(n, 256)
    grid = (b, n // tile)
    outT = pl.pallas_call(
        functools.partial(_unpool_kernel, m=m, tile=tile),
        grid=grid,
        in_specs=[
            pl.BlockSpec((1, tile, m), lambda bi, ti: (bi, ti, 0)),
            pl.BlockSpec((1, m, c), lambda bi, ti: (bi, 0, 0)),
        ],
        out_specs=pl.BlockSpec((1, tile, c), lambda bi, ti: (bi, ti, 0)),
        out_shape=jax.ShapeDtypeStruct((b, n, c), jnp.float32),
    )(neg, featT)
    return jnp.transpose(outT, (0, 2, 1))


def _bn(y, g, b):
    axes = (0,) + tuple(range(2, y.ndim))
    mean = jnp.mean(y, axis=axes, keepdims=True)
    var = jnp.var(y, axis=axes, keepdims=True)
    yn = (y - mean) / jnp.sqrt(var + 1e-5)
    shape = [1, -1] + [1] * (y.ndim - 2)
    return yn * g.reshape(shape) + b.reshape(shape)


def _lrelu(y):
    return jnp.where(y >= 0, y, 0.2 * y)


def _conv2(y, W, g, b):
    return _lrelu(_bn(jnp.einsum('oc,bcnk->bonk', W, y), g, b))


def _conv1(y, W, g, b):
    return _lrelu(_bn(jnp.einsum('oc,bcn->bon', W, y), g, b))


def kernel(x, l, params):
    p = params
    xyz = x
    xyz8 = jnp.pad(xyz, ((0, 0), (0, 5), (0, 0)))

    y = _graph_feature(x, K)
    y = _conv2(y, p['conv1'], p['bn1_g'], p['bn1_b'])
    y = _conv2(y, p['conv2'], p['bn2_g'], p['bn2_b'])
    x1 = jnp.max(y, axis=-1)

    node1_8, nf1 = _pool_pallas(xyz8, x1, p['pool1'], N // 4)
    node1 = node1_8[:, :3, :]

    y = _graph_feature(nf1, K // 2)
    y = _conv2(y, p['conv3'], p['bn3_g'], p['bn3_b'])
    y = _conv2(y, p['conv4'], p['bn4_g'], p['bn4_b'])
    x2 = jnp.max(y, axis=-1)

    node2_8, nf2 = _pool_pallas(node1_8, x2, p['pool2'], N // 16)
    node2 = node2_8[:, :3, :]

    y = _graph_feature(nf2, K // 4)
    y = _conv2(y, p['conv5'], p['bn5_g'], p['bn5_b'])
    x3 = jnp.max(y, axis=-1)

    node3_8, nf3 = _pool_pallas(node2_8, x3, p['pool3'], N // 64)
    node3 = node3_8[:, :3, :]

    y = _graph_feature(nf3, K // 8)
    y = _conv2(y, p['conv6'], p['bn6_g'], p['bn6_b'])
    x4 = jnp.max(y, axis=-1)

    x1_t = jnp.max(x1, axis=-1, keepdims=True)
    x2_t = jnp.max(x2, axis=-1, keepdims=True)
    x3_t = jnp.max(x3, axis=-1, keepdims=True)
    x4_t = jnp.max(x4, axis=-1, keepdims=True)
    y = jnp.concatenate([x1_t, x2_t, x3_t, x4_t], axis=1)
    y = _conv1(y, p['conv6_m'], p['bn6_m_g'], p['bn6_m_b'])
    lv = l.reshape(l.shape[0], -1, 1)
    lv = _conv1(lv, p['conv7'], p['bn7_g'], p['bn7_b'])
    y = jnp.concatenate([y, lv], axis=1)
    y = _conv1(y, p['conv8'], p['bn8_g'], p['bn8_b'])
    y = jnp.tile(y, (1, 1, x4.shape[-1]))
    y = jnp.concatenate([y, x4], axis=1)
    y = _conv1(y, p['conv9'], p['bn9_g'], p['bn9_b'])

    node1s = node1_8[:, :3, :]
    node2s = node2_8[:, :3, :]
    node3s = node3_8[:, :3, :]
    y = _unpool(node3s, node2s, y)
    y = jnp.concatenate([y, x3], axis=1)
    y = _conv1(y, p['conv10'], p['bn10_g'], p['bn10_b'])

    y = _unpool(node2s, node1s, y)
    y = jnp.concatenate([y, x2], axis=1)
    y = _conv1(y, p['conv11'], p['bn11_g'], p['bn11_b'])

    y = _unpool(node1s, xyz, y)
    y = jnp.concatenate([y, x1], axis=1)
    y = _conv1(y, p['conv12'], p['bn12_g'], p['bn12_b'])
    y = jnp.einsum('oc,bcn->bon', p['conv13'], y)
    return (y, node1, node2, node3, node1, node2)
